# baseline (speedup 1.0000x reference)
import functools

import jax
import jax.numpy as jnp
from jax.experimental import pallas as pl

_HIGHEST = jax.lax.Precision.HIGHEST
_INT_MIN = -2147483648
_INT_MAX = 2147483647
_MAG = 0x7FFFFFFF


def _dotT(a, b, precision=_HIGHEST):
    return jax.lax.dot_general(a, b, (((1,), (1,)), ((), ())),
                               preferred_element_type=jnp.float32,
                               precision=precision)


def _dot(a, b, precision=_HIGHEST):
    return jax.lax.dot_general(a, b, (((1,), (0,)), ((), ())),
                               preferred_element_type=jnp.float32,
                               precision=precision)


def _monotone_int(x):
    b = jax.lax.bitcast_convert_type(x, jnp.int32)
    return jnp.where(b < 0, -(b & _MAG), b)


def _kth_smallest_blocks(key_block_fn, nblk, tb, k):
    lo0 = jnp.full((tb, 1), _INT_MIN, jnp.int32)
    hi0 = jnp.full((tb, 1), _INT_MAX, jnp.int32)
    kf = jnp.float32(k)

    def body(_, carry):
        lo, hi = carry
        mid = (lo >> 1) + (hi >> 1) + (lo & hi & 1)

        def cnt_body(j, cnt):
            return cnt + jnp.sum((key_block_fn(j) <= mid).astype(jnp.float32),
                                 axis=1, keepdims=True)

        cnt = jax.lax.fori_loop(0, nblk, cnt_body,
                                jnp.zeros((tb, 1), jnp.float32))
        ge = cnt >= kf
        return jnp.where(ge, lo, mid + 1), jnp.where(ge, mid, hi)

    lo, hi = jax.lax.fori_loop(0, 32, body, (lo0, hi0))
    return hi


def _eg(s, tau, sel):
    raw = s - tau
    g = jnp.where(raw > 0, raw, 1e-8 * jnp.exp(jnp.minimum(raw, 0.0)))
    eg = jnp.exp(g) - 1.0
    return jnp.where(sel, eg, 0.0)


def _thresholds(s_ref, d_ref, nbw, nblk, tb, n_cand, max_k):
    t64 = _kth_smallest_blocks(
        lambda j: _monotone_int(d_ref[:, pl.ds(j * nbw, nbw)]),
        nblk, tb, n_cand)

    def skey(j):
        sl = pl.ds(j * nbw, nbw)
        cand = _monotone_int(d_ref[:, sl]) <= t64
        return jnp.where(cand, -_monotone_int(s_ref[:, sl]),
                         jnp.int32(_INT_MAX))

    t16 = _kth_smallest_blocks(skey, nblk, tb, max_k)
    return t64, t16


def _sel_of(d_blk, s_blk, t64, t16):
    return (_monotone_int(d_blk) <= t64) & (-_monotone_int(s_blk) <= t16)


def _stage_a1_body(x_ref, qkpos_ref, vpos_ref, qkn_ref, vn_ref,
                   nposqk_ref, nposv_ref,
                   sqk_ref, sv_ref, dqk_ref, dv_ref):
    x = x_ref[...]
    qkpos = qkpos_ref[...]
    vpos = vpos_ref[...]
    qkn = qkn_ref[...]
    vn = vn_ref[...]
    npq = nposqk_ref[...]
    npv = nposv_ref[...]
    sqk_ref[...] = _dotT(x, qkn)
    sv_ref[...] = _dotT(x, vn)
    psq_qk = jnp.sum(qkpos * qkpos, axis=1, keepdims=True)
    psq_v = jnp.sum(vpos * vpos, axis=1, keepdims=True)
    nsq_q = jnp.sum(npq * npq, axis=1)[None, :]
    nsq_v = jnp.sum(npv * npv, axis=1)[None, :]
    dqk_ref[...] = nsq_q - 2.0 * _dotT(qkpos, npq) + psq_qk
    dv_ref[...] = nsq_v - 2.0 * _dotT(vpos, npv) + psq_v


def _stage_a2qk_body(tb, nbw, nblk, n_cand, max_k,
                     s_ref, d_ref, tauq_ref, tauk_ref, qkn_ref,
                     q_ref, k_ref, pos_ref):
    t64, t16 = _thresholds(s_ref, d_ref, nbw, nblk, tb, n_cand, max_k)
    tau_q = tauq_ref[...]
    tau_k = tauk_ref[...]

    def p3a(j, carry):
        gs_q, gm_q, gs_k, gm_k, pos_q = carry
        sl = pl.ds(j * nbw, nbw)
        s = s_ref[:, sl]
        d = d_ref[:, sl]
        sel = _sel_of(d, s, t64, t16)
        eq = _eg(s, tau_q, sel)
        ek = _eg(s, tau_k, sel)
        gs_q += jnp.sum(eq, axis=1, keepdims=True)
        gm_q = jnp.maximum(gm_q, jnp.max(eq, axis=1, keepdims=True))
        gs_k += jnp.sum(ek, axis=1, keepdims=True)
        gm_k = jnp.maximum(gm_k, jnp.max(ek, axis=1, keepdims=True))
        pos_q += jnp.sum(eq * d, axis=1, keepdims=True)
        return (gs_q, gm_q, gs_k, gm_k, pos_q)

    z = jnp.zeros((tb, 1), jnp.float32)
    gs_q, gm_q, gs_k, gm_k, pos_q = jax.lax.fori_loop(
        0, nblk, p3a, (z, z, z, z, z))
    sc_q = jnp.tanh(gm_q) / (gs_q + 1e-8)
    sc_k = jnp.tanh(gm_k) / (gs_k + 1e-8)

    q_ref[...] = jnp.zeros_like(q_ref)
    k_ref[...] = jnp.zeros_like(k_ref)

    def p3b(j, _):
        sl = pl.ds(j * nbw, nbw)
        s = s_ref[:, sl]
        d = d_ref[:, sl]
        sel = _sel_of(d, s, t64, t16)
        qkn = qkn_ref[sl, :]
        q_ref[...] += _dot(s * _eg(s, tau_q, sel), qkn)
        k_ref[...] += _dot(s * _eg(s, tau_k, sel), qkn)
        return 0

    jax.lax.fori_loop(0, nblk, p3b, 0)
    q_ref[...] *= sc_q
    k_ref[...] *= sc_k
    pos_ref[...] = jnp.sum(sc_q * pos_q)[None, None, None]


def _stage_a2v_body(tb, nbw, nblk, n_cand, max_k,
                    s_ref, d_ref, tauv_ref, vn_ref, v_ref, pos_ref):
    t64, t16 = _thresholds(s_ref, d_ref, nbw, nblk, tb, n_cand, max_k)
    tau_v = tauv_ref[...]

    def p3a(j, carry):
        gs_v, gm_v, pos_v = carry
        sl = pl.ds(j * nbw, nbw)
        s = s_ref[:, sl]
        d = d_ref[:, sl]
        sel = _sel_of(d, s, t64, t16)
        ev = _eg(s, tau_v, sel)
        gs_v += jnp.sum(ev, axis=1, keepdims=True)
        gm_v = jnp.maximum(gm_v, jnp.max(ev, axis=1, keepdims=True))
        pos_v += jnp.sum(ev * d, axis=1, keepdims=True)
        return (gs_v, gm_v, pos_v)

    z = jnp.zeros((tb, 1), jnp.float32)
    gs_v, gm_v, pos_v = jax.lax.fori_loop(0, nblk, p3a, (z, z, z))
    sc_v = jnp.tanh(gm_v) / (gs_v + 1e-8)

    v_ref[...] = jnp.zeros_like(v_ref)

    def p3b(j, _):
        sl = pl.ds(j * nbw, nbw)
        s = s_ref[:, sl]
        d = d_ref[:, sl]
        sel = _sel_of(d, s, t64, t16)
        v_ref[...] += _dot(s * _eg(s, tau_v, sel), vn_ref[sl, :])
        return 0

    jax.lax.fori_loop(0, nblk, p3b, 0)
    v_ref[...] *= sc_v
    pos_ref[...] = jnp.sum(sc_v * pos_v)[None, None, None]


def _stage_b_body(scale, tbq, x_ref, k_ref, v_ref, o_ref):
    i = pl.program_id(1)
    q = x_ref[0]
    k = k_ref[0]
    v = v_ref[0]
    s = _dotT(q, k) * (1.0 / scale)
    row = jax.lax.broadcasted_iota(jnp.int32, s.shape, 0) + i * tbq
    col = jax.lax.broadcasted_iota(jnp.int32, s.shape, 1)
    s = jnp.where(col <= row, s, jnp.finfo(jnp.float32).min)
    m = jnp.max(s, axis=1, keepdims=True)
    p = jnp.exp(s - m)
    l = jnp.sum(p, axis=1, keepdims=True)
    o_ref[...] = (_dot(p, v) / l)[None]


def _stage_c_body(a_ref, w_ref, o_ref):
    o_ref[...] = _dot(a_ref[...], w_ref[...])


def kernel(x, qk_pos, v_pos, tau_Q, tau_K, tau_V, qk_neurons, v_neurons,
           npos_qk, npos_v, expand_O_kernel):
    B, S, D = x.shape
    N, _ = qk_neurons.shape
    P = qk_pos.shape[-1]
    n_heads = 16
    dh = D // n_heads
    max_k = 16
    n_cand = max_k * 4

    x2 = x.reshape(S, D)
    qkp = qk_pos.reshape(S, P)
    vp = v_pos.reshape(S, P)
    tq = tau_Q.reshape(S, 1)
    tk = tau_K.reshape(S, 1)
    tv = tau_V.reshape(S, 1)

    tb = 128
    nb = S // tb
    nbw = 512
    nblk = N // nbw

    tok = lambda i: (i, 0)
    full = lambda i: (0, 0)

    sqk, sv, dqk, dv = pl.pallas_call(
        _stage_a1_body,
        grid=(nblk, nb),
        in_specs=[
            pl.BlockSpec((tb, D), lambda j, i: (i, 0)),
            pl.BlockSpec((tb, P), lambda j, i: (i, 0)),
            pl.BlockSpec((tb, P), lambda j, i: (i, 0)),
            pl.BlockSpec((nbw, D), lambda j, i: (j, 0)),
            pl.BlockSpec((nbw, D), lambda j, i: (j, 0)),
            pl.BlockSpec((nbw, P), lambda j, i: (j, 0)),
            pl.BlockSpec((nbw, P), lambda j, i: (j, 0)),
        ],
        out_specs=[
            pl.BlockSpec((tb, nbw), lambda j, i: (i, j)),
            pl.BlockSpec((tb, nbw), lambda j, i: (i, j)),
            pl.BlockSpec((tb, nbw), lambda j, i: (i, j)),
            pl.BlockSpec((tb, nbw), lambda j, i: (i, j)),
        ],
        out_shape=[
            jax.ShapeDtypeStruct((S, N), jnp.float32),
            jax.ShapeDtypeStruct((S, N), jnp.float32),
            jax.ShapeDtypeStruct((S, N), jnp.float32),
            jax.ShapeDtypeStruct((S, N), jnp.float32),
        ],
    )(x2, qkp, vp, qk_neurons, v_neurons, npos_qk, npos_v)

    q_buf, k_buf, pos_qk = pl.pallas_call(
        functools.partial(_stage_a2qk_body, tb, nbw, nblk, n_cand, max_k),
        grid=(nb,),
        in_specs=[
            pl.BlockSpec((tb, N), tok),
            pl.BlockSpec((tb, N), tok),
            pl.BlockSpec((tb, 1), tok),
            pl.BlockSpec((tb, 1), tok),
            pl.BlockSpec((N, D), full),
        ],
        out_specs=[
            pl.BlockSpec((tb, D), tok),
            pl.BlockSpec((tb, D), tok),
            pl.BlockSpec((1, 1, 1), lambda i: (i, 0, 0)),
        ],
        out_shape=[
            jax.ShapeDtypeStruct((S, D), jnp.float32),
            jax.ShapeDtypeStruct((S, D), jnp.float32),
            jax.ShapeDtypeStruct((nb, 1, 1), jnp.float32),
        ],
    )(sqk, dqk, tq, tk, qk_neurons)

    v_buf, pos_v = pl.pallas_call(
        functools.partial(_stage_a2v_body, tb, nbw, nblk, n_cand, max_k),
        grid=(nb,),
        in_specs=[
            pl.BlockSpec((tb, N), tok),
            pl.BlockSpec((tb, N), tok),
            pl.BlockSpec((tb, 1), tok),
            pl.BlockSpec((N, D), full),
        ],
        out_specs=[
            pl.BlockSpec((tb, D), tok),
            pl.BlockSpec((1, 1, 1), lambda i: (i, 0, 0)),
        ],
        out_shape=[
            jax.ShapeDtypeStruct((S, D), jnp.float32),
            jax.ShapeDtypeStruct((nb, 1, 1), jnp.float32),
        ],
    )(sv, dv, tv, v_neurons)

    tbq = 128
    nbq = S // tbq
    scale = float(dh) ** 0.5
    q3 = q_buf.reshape(S, n_heads, dh).transpose(1, 0, 2)
    k3 = k_buf.reshape(S, n_heads, dh).transpose(1, 0, 2)
    v3 = v_buf.reshape(S, n_heads, dh).transpose(1, 0, 2)
    attn = pl.pallas_call(
        functools.partial(_stage_b_body, scale, tbq),
        grid=(n_heads, nbq),
        in_specs=[
            pl.BlockSpec((1, tbq, dh), lambda h, i: (h, i, 0)),
            pl.BlockSpec((1, S, dh), lambda h, i: (h, 0, 0)),
            pl.BlockSpec((1, S, dh), lambda h, i: (h, 0, 0)),
        ],
        out_specs=pl.BlockSpec((1, tbq, dh), lambda h, i: (h, i, 0)),
        out_shape=jax.ShapeDtypeStruct((n_heads, S, dh), jnp.float32),
    )(q3, k3, v3)
    attn = attn.transpose(1, 0, 2).reshape(S, D)

    out = pl.pallas_call(
        _stage_c_body,
        grid=(nb,),
        in_specs=[
            pl.BlockSpec((tb, D), tok),
            pl.BlockSpec((D, D), full),
        ],
        out_specs=pl.BlockSpec((tb, D), tok),
        out_shape=jax.ShapeDtypeStruct((S, D), jnp.float32),
    )(attn, expand_O_kernel)

    pos_loss = (jnp.sum(pos_qk) + jnp.sum(pos_v)) / jnp.float32(S * n_cand)
    return out.reshape(B, S, D), pos_loss


# speedup vs baseline: 1.3192x; 1.3192x over previous
import functools

import jax
import jax.numpy as jnp
from jax.experimental import pallas as pl
from jax.experimental.pallas import tpu as pltpu

_HIGHEST = jax.lax.Precision.HIGHEST
_HIGH = jax.lax.Precision.HIGH
_DEFAULT = jax.lax.Precision.DEFAULT
_INT_MIN = -2147483648
_INT_MAX = 2147483647
_MAG = 0x7FFFFFFF


def _dotT(a, b, precision=_HIGHEST):
    return jax.lax.dot_general(a, b, (((1,), (1,)), ((), ())),
                               preferred_element_type=jnp.float32,
                               precision=precision)


def _dot(a, b, precision=_HIGHEST):
    return jax.lax.dot_general(a, b, (((1,), (0,)), ((), ())),
                               preferred_element_type=jnp.float32,
                               precision=precision)


def _monotone_int(x):
    b = jax.lax.bitcast_convert_type(x, jnp.int32)
    return jnp.where(b < 0, -(b & _MAG), b)


def _kth_smallest_blocks(key_block_fn, nblk, tb, k, lo0, hi0):
    kf = jnp.float32(k)

    def cond(carry):
        i, lo, hi = carry
        return (i < 32) & jnp.any(lo < hi)

    def body(carry):
        i, lo, hi = carry
        mid = (lo >> 1) + (hi >> 1) + (lo & hi & 1)

        def cnt_body(j, cnt):
            return cnt + jnp.sum((key_block_fn(j) <= mid).astype(jnp.float32),
                                 axis=1, keepdims=True)

        cnt = jax.lax.fori_loop(0, nblk, cnt_body,
                                jnp.zeros((tb, 1), jnp.float32))
        ge = cnt >= kf
        return i + 1, jnp.where(ge, lo, mid + 1), jnp.where(ge, mid, hi)

    _, lo, hi = jax.lax.while_loop(cond, body, (jnp.int32(0), lo0, hi0))
    return hi


def _eg(s, tau, sel):
    raw = s - tau
    g = jnp.where(raw > 0, raw, 1e-8 * jnp.exp(jnp.minimum(raw, 0.0)))
    eg = jnp.exp(g) - 1.0
    return jnp.where(sel, eg, 0.0)


def _thresholds(s_ref, d_ref, dkey_ref, skey_ref, nbw, nblk, tb,
                n_cand, max_k):
    big = jnp.full((tb, 1), _INT_MAX, jnp.int32)
    small = jnp.full((tb, 1), _INT_MIN, jnp.int32)

    def prep_d(j, carry):
        lo, hi = carry
        sl = pl.ds(j * nbw, nbw)
        key = _monotone_int(d_ref[:, sl])
        dkey_ref[:, sl] = key
        lo = jnp.minimum(lo, jnp.min(key, axis=1, keepdims=True))
        hi = jnp.maximum(hi, jnp.max(key, axis=1, keepdims=True))
        return lo, hi

    dlo, dhi = jax.lax.fori_loop(0, nblk, prep_d, (big, small))
    t64 = _kth_smallest_blocks(
        lambda j: dkey_ref[:, pl.ds(j * nbw, nbw)], nblk, tb, n_cand,
        dlo, dhi)

    def prep_s(j, carry):
        lo, hi = carry
        sl = pl.ds(j * nbw, nbw)
        cand = dkey_ref[:, sl] <= t64
        key = jnp.where(cand, -_monotone_int(s_ref[:, sl]),
                        jnp.int32(_INT_MAX))
        skey_ref[:, sl] = key
        lo = jnp.minimum(lo, jnp.min(key, axis=1, keepdims=True))
        hi = jnp.maximum(hi, jnp.max(jnp.where(cand, key, _INT_MIN),
                                     axis=1, keepdims=True))
        return lo, hi

    slo, shi = jax.lax.fori_loop(0, nblk, prep_s, (big, small))
    t16 = _kth_smallest_blocks(
        lambda j: skey_ref[:, pl.ds(j * nbw, nbw)], nblk, tb, max_k,
        slo, shi)
    return t64, t16


def _stage_a1_body(x_ref, qkpos_ref, vpos_ref, qkn_ref, vn_ref,
                   nposqk_ref, nposv_ref,
                   sqk_ref, sv_ref, dqk_ref, dv_ref):
    x = x_ref[...]
    qkpos = qkpos_ref[...]
    vpos = vpos_ref[...]
    qkn = qkn_ref[...]
    vn = vn_ref[...]
    npq = nposqk_ref[...]
    npv = nposv_ref[...]
    sqk_ref[...] = _dotT(x, qkn, _DEFAULT)
    sv_ref[...] = _dotT(x, vn, _DEFAULT)
    psq_qk = jnp.sum(qkpos * qkpos, axis=1, keepdims=True)
    psq_v = jnp.sum(vpos * vpos, axis=1, keepdims=True)
    nsq_q = jnp.sum(npq * npq, axis=1)[None, :]
    nsq_v = jnp.sum(npv * npv, axis=1)[None, :]
    dqk_ref[...] = nsq_q - 2.0 * _dotT(qkpos, npq, _HIGHEST) + psq_qk
    dv_ref[...] = nsq_v - 2.0 * _dotT(vpos, npv, _HIGHEST) + psq_v


def _stage_a2qk_body(tb, nbw, nblk, n_cand, max_k,
                     s_ref, d_ref, tauq_ref, tauk_ref, qkn_ref,
                     q_ref, k_ref, pos_ref, dkey_ref, skey_ref):
    t64, t16 = _thresholds(s_ref, d_ref, dkey_ref, skey_ref,
                           nbw, nblk, tb, n_cand, max_k)
    tau_q = tauq_ref[...]
    tau_k = tauk_ref[...]

    def p3a(j, carry):
        gs_q, gm_q, gs_k, gm_k, pos_q = carry
        sl = pl.ds(j * nbw, nbw)
        s = s_ref[:, sl]
        d = d_ref[:, sl]
        sel = skey_ref[:, sl] <= t16
        eq = _eg(s, tau_q, sel)
        ek = _eg(s, tau_k, sel)
        gs_q += jnp.sum(eq, axis=1, keepdims=True)
        gm_q = jnp.maximum(gm_q, jnp.max(eq, axis=1, keepdims=True))
        gs_k += jnp.sum(ek, axis=1, keepdims=True)
        gm_k = jnp.maximum(gm_k, jnp.max(ek, axis=1, keepdims=True))
        pos_q += jnp.sum(eq * d, axis=1, keepdims=True)
        return (gs_q, gm_q, gs_k, gm_k, pos_q)

    z = jnp.zeros((tb, 1), jnp.float32)
    gs_q, gm_q, gs_k, gm_k, pos_q = jax.lax.fori_loop(
        0, nblk, p3a, (z, z, z, z, z))
    sc_q = jnp.tanh(gm_q) / (gs_q + 1e-8)
    sc_k = jnp.tanh(gm_k) / (gs_k + 1e-8)

    q_ref[...] = jnp.zeros_like(q_ref)
    k_ref[...] = jnp.zeros_like(k_ref)

    def p3b(j, _):
        sl = pl.ds(j * nbw, nbw)
        s = s_ref[:, sl]
        sel = skey_ref[:, sl] <= t16
        qkn = qkn_ref[sl, :]
        q_ref[...] += _dot(s * _eg(s, tau_q, sel), qkn, _DEFAULT)
        k_ref[...] += _dot(s * _eg(s, tau_k, sel), qkn, _DEFAULT)
        return 0

    jax.lax.fori_loop(0, nblk, p3b, 0)
    q_ref[...] *= sc_q
    k_ref[...] *= sc_k
    pos_ref[...] = jnp.sum(sc_q * pos_q)[None, None, None]


def _stage_a2v_body(tb, nbw, nblk, n_cand, max_k,
                    s_ref, d_ref, tauv_ref, vn_ref, v_ref, pos_ref,
                    dkey_ref, skey_ref):
    t64, t16 = _thresholds(s_ref, d_ref, dkey_ref, skey_ref,
                           nbw, nblk, tb, n_cand, max_k)
    tau_v = tauv_ref[...]

    def p3a(j, carry):
        gs_v, gm_v, pos_v = carry
        sl = pl.ds(j * nbw, nbw)
        s = s_ref[:, sl]
        d = d_ref[:, sl]
        sel = skey_ref[:, sl] <= t16
        ev = _eg(s, tau_v, sel)
        gs_v += jnp.sum(ev, axis=1, keepdims=True)
        gm_v = jnp.maximum(gm_v, jnp.max(ev, axis=1, keepdims=True))
        pos_v += jnp.sum(ev * d, axis=1, keepdims=True)
        return (gs_v, gm_v, pos_v)

    z = jnp.zeros((tb, 1), jnp.float32)
    gs_v, gm_v, pos_v = jax.lax.fori_loop(0, nblk, p3a, (z, z, z))
    sc_v = jnp.tanh(gm_v) / (gs_v + 1e-8)

    v_ref[...] = jnp.zeros_like(v_ref)

    def p3b(j, _):
        sl = pl.ds(j * nbw, nbw)
        s = s_ref[:, sl]
        sel = skey_ref[:, sl] <= t16
        v_ref[...] += _dot(s * _eg(s, tau_v, sel), vn_ref[sl, :], _DEFAULT)
        return 0

    jax.lax.fori_loop(0, nblk, p3b, 0)
    v_ref[...] *= sc_v
    pos_ref[...] = jnp.sum(sc_v * pos_v)[None, None, None]


def _stage_b_body(scale, tbq, x_ref, k_ref, v_ref, o_ref):
    i = pl.program_id(1)
    q = x_ref[0]
    k = k_ref[0]
    v = v_ref[0]
    s = _dotT(q, k, _DEFAULT) * (1.0 / scale)
    row = jax.lax.broadcasted_iota(jnp.int32, s.shape, 0) + i * tbq
    col = jax.lax.broadcasted_iota(jnp.int32, s.shape, 1)
    s = jnp.where(col <= row, s, jnp.finfo(jnp.float32).min)
    m = jnp.max(s, axis=1, keepdims=True)
    p = jnp.exp(s - m)
    l = jnp.sum(p, axis=1, keepdims=True)
    o_ref[...] = (_dot(p, v, _DEFAULT) / l)[None]


def _stage_c_body(a_ref, w_ref, o_ref):
    o_ref[...] = _dot(a_ref[...], w_ref[...], _DEFAULT)


def kernel(x, qk_pos, v_pos, tau_Q, tau_K, tau_V, qk_neurons, v_neurons,
           npos_qk, npos_v, expand_O_kernel):
    B, S, D = x.shape
    N, _ = qk_neurons.shape
    P = qk_pos.shape[-1]
    n_heads = 16
    dh = D // n_heads
    max_k = 16
    n_cand = max_k * 4

    x2 = x.reshape(S, D)
    qkp = qk_pos.reshape(S, P)
    vp = v_pos.reshape(S, P)
    tq = tau_Q.reshape(S, 1)
    tk = tau_K.reshape(S, 1)
    tv = tau_V.reshape(S, 1)

    tb = 128
    nb = S // tb
    nbw = 512
    nblk = N // nbw

    tok = lambda i: (i, 0)
    full = lambda i: (0, 0)

    sqk, sv, dqk, dv = pl.pallas_call(
        _stage_a1_body,
        grid=(nblk, nb),
        in_specs=[
            pl.BlockSpec((tb, D), lambda j, i: (i, 0)),
            pl.BlockSpec((tb, P), lambda j, i: (i, 0)),
            pl.BlockSpec((tb, P), lambda j, i: (i, 0)),
            pl.BlockSpec((nbw, D), lambda j, i: (j, 0)),
            pl.BlockSpec((nbw, D), lambda j, i: (j, 0)),
            pl.BlockSpec((nbw, P), lambda j, i: (j, 0)),
            pl.BlockSpec((nbw, P), lambda j, i: (j, 0)),
        ],
        out_specs=[
            pl.BlockSpec((tb, nbw), lambda j, i: (i, j)),
            pl.BlockSpec((tb, nbw), lambda j, i: (i, j)),
            pl.BlockSpec((tb, nbw), lambda j, i: (i, j)),
            pl.BlockSpec((tb, nbw), lambda j, i: (i, j)),
        ],
        out_shape=[
            jax.ShapeDtypeStruct((S, N), jnp.float32),
            jax.ShapeDtypeStruct((S, N), jnp.float32),
            jax.ShapeDtypeStruct((S, N), jnp.float32),
            jax.ShapeDtypeStruct((S, N), jnp.float32),
        ],
    )(x2, qkp, vp, qk_neurons, v_neurons, npos_qk, npos_v)

    q_buf, k_buf, pos_qk = pl.pallas_call(
        functools.partial(_stage_a2qk_body, tb, nbw, nblk, n_cand, max_k),
        grid=(nb,),
        in_specs=[
            pl.BlockSpec((tb, N), tok),
            pl.BlockSpec((tb, N), tok),
            pl.BlockSpec((tb, 1), tok),
            pl.BlockSpec((tb, 1), tok),
            pl.BlockSpec((N, D), full),
        ],
        out_specs=[
            pl.BlockSpec((tb, D), tok),
            pl.BlockSpec((tb, D), tok),
            pl.BlockSpec((1, 1, 1), lambda i: (i, 0, 0)),
        ],
        out_shape=[
            jax.ShapeDtypeStruct((S, D), jnp.float32),
            jax.ShapeDtypeStruct((S, D), jnp.float32),
            jax.ShapeDtypeStruct((nb, 1, 1), jnp.float32),
        ],
        scratch_shapes=[
            pltpu.VMEM((tb, N), jnp.int32),
            pltpu.VMEM((tb, N), jnp.int32),
        ],
    )(sqk, dqk, tq, tk, qk_neurons)

    v_buf, pos_v = pl.pallas_call(
        functools.partial(_stage_a2v_body, tb, nbw, nblk, n_cand, max_k),
        grid=(nb,),
        in_specs=[
            pl.BlockSpec((tb, N), tok),
            pl.BlockSpec((tb, N), tok),
            pl.BlockSpec((tb, 1), tok),
            pl.BlockSpec((N, D), full),
        ],
        out_specs=[
            pl.BlockSpec((tb, D), tok),
            pl.BlockSpec((1, 1, 1), lambda i: (i, 0, 0)),
        ],
        out_shape=[
            jax.ShapeDtypeStruct((S, D), jnp.float32),
            jax.ShapeDtypeStruct((nb, 1, 1), jnp.float32),
        ],
        scratch_shapes=[
            pltpu.VMEM((tb, N), jnp.int32),
            pltpu.VMEM((tb, N), jnp.int32),
        ],
    )(sv, dv, tv, v_neurons)

    tbq = 128
    nbq = S // tbq
    scale = float(dh) ** 0.5
    q3 = q_buf.reshape(S, n_heads, dh).transpose(1, 0, 2)
    k3 = k_buf.reshape(S, n_heads, dh).transpose(1, 0, 2)
    v3 = v_buf.reshape(S, n_heads, dh).transpose(1, 0, 2)
    attn = pl.pallas_call(
        functools.partial(_stage_b_body, scale, tbq),
        grid=(n_heads, nbq),
        in_specs=[
            pl.BlockSpec((1, tbq, dh), lambda h, i: (h, i, 0)),
            pl.BlockSpec((1, S, dh), lambda h, i: (h, 0, 0)),
            pl.BlockSpec((1, S, dh), lambda h, i: (h, 0, 0)),
        ],
        out_specs=pl.BlockSpec((1, tbq, dh), lambda h, i: (h, i, 0)),
        out_shape=jax.ShapeDtypeStruct((n_heads, S, dh), jnp.float32),
    )(q3, k3, v3)
    attn = attn.transpose(1, 0, 2).reshape(S, D)

    out = pl.pallas_call(
        _stage_c_body,
        grid=(nb,),
        in_specs=[
            pl.BlockSpec((tb, D), tok),
            pl.BlockSpec((D, D), full),
        ],
        out_specs=pl.BlockSpec((tb, D), tok),
        out_shape=jax.ShapeDtypeStruct((S, D), jnp.float32),
    )(attn, expand_O_kernel)

    pos_loss = (jnp.sum(pos_qk) + jnp.sum(pos_v)) / jnp.float32(S * n_cand)
    return out.reshape(B, S, D), pos_loss


# speedup vs baseline: 1.4922x; 1.1311x over previous
import functools

import jax
import jax.numpy as jnp
from jax import lax
from jax.experimental import pallas as pl
from jax.experimental.pallas import tpu as pltpu
from jax.experimental.pallas import tpu_sc as plsc

_HIGHEST = jax.lax.Precision.HIGHEST
_HIGH = jax.lax.Precision.HIGH
_DEFAULT = jax.lax.Precision.DEFAULT
_INT_MIN = -2147483648
_INT_MAX = 2147483647
_MAG = 0x7FFFFFFF


def _dotT(a, b, precision=_HIGHEST):
    return jax.lax.dot_general(a, b, (((1,), (1,)), ((), ())),
                               preferred_element_type=jnp.float32,
                               precision=precision)


def _dot(a, b, precision=_HIGHEST):
    return jax.lax.dot_general(a, b, (((1,), (0,)), ((), ())),
                               preferred_element_type=jnp.float32,
                               precision=precision)


def _monotone_int(x):
    b = jax.lax.bitcast_convert_type(x, jnp.int32)
    return jnp.where(b < 0, -(b & _MAG), b)


def _kth_smallest_blocks(key_block_fn, nblk, tb, k, lo0, hi0):
    kf = jnp.float32(k)

    def cond(carry):
        i, lo, hi = carry
        return (i < 32) & jnp.any(lo < hi)

    def body(carry):
        i, lo, hi = carry
        mid = (lo >> 1) + (hi >> 1) + (lo & hi & 1)

        def cnt_body(j, cnt):
            return cnt + jnp.sum((key_block_fn(j) <= mid).astype(jnp.float32),
                                 axis=1, keepdims=True)

        cnt = jax.lax.fori_loop(0, nblk, cnt_body,
                                jnp.zeros((tb, 1), jnp.float32))
        ge = cnt >= kf
        return i + 1, jnp.where(ge, lo, mid + 1), jnp.where(ge, mid, hi)

    _, lo, hi = jax.lax.while_loop(cond, body, (jnp.int32(0), lo0, hi0))
    return hi


def _eg(s, tau, sel):
    raw = s - tau
    g = jnp.where(raw > 0, raw, 1e-8 * jnp.exp(jnp.minimum(raw, 0.0)))
    eg = jnp.exp(g) - 1.0
    return jnp.where(sel, eg, 0.0)


def _thresholds_compact(cd_ref, cs_ref, tb, n_cand, max_k):
    dkey = _monotone_int(cd_ref[...])
    dlo = jnp.min(dkey, axis=1, keepdims=True)
    dhi = jnp.max(dkey, axis=1, keepdims=True)
    t64 = _kth_smallest_blocks(lambda j: dkey, 1, tb, n_cand, dlo, dhi)

    cand = dkey <= t64
    skey = jnp.where(cand, -_monotone_int(cs_ref[...]), jnp.int32(_INT_MAX))
    slo = jnp.min(skey, axis=1, keepdims=True)
    shi = jnp.max(jnp.where(cand, skey, _INT_MIN), axis=1, keepdims=True)
    t16 = _kth_smallest_blocks(lambda j: skey, 1, tb, max_k, slo, shi)
    return t64, t16


def _sel_of(d_blk, s_blk, t64, t16):
    return (_monotone_int(d_blk) <= t64) & (-_monotone_int(s_blk) <= t16)


def _stage_a1_body(x_ref, qkpos_ref, vpos_ref, qkn_ref, vn_ref,
                   nposqk_ref, nposv_ref,
                   sqk_ref, sv_ref, dqk_ref, dv_ref):
    x = x_ref[...]
    qkpos = qkpos_ref[...]
    vpos = vpos_ref[...]
    qkn = qkn_ref[...]
    vn = vn_ref[...]
    npq = nposqk_ref[...]
    npv = nposv_ref[...]
    sqk_ref[...] = _dotT(x, qkn, _DEFAULT)
    sv_ref[...] = _dotT(x, vn, _DEFAULT)
    psq_qk = jnp.sum(qkpos * qkpos, axis=1, keepdims=True)
    psq_v = jnp.sum(vpos * vpos, axis=1, keepdims=True)
    nsq_q = jnp.sum(npq * npq, axis=1)[None, :]
    nsq_v = jnp.sum(npv * npv, axis=1)[None, :]
    dqk_ref[...] = nsq_q - 2.0 * _dotT(qkpos, npq, _HIGHEST) + psq_qk
    dv_ref[...] = nsq_v - 2.0 * _dotT(vpos, npv, _HIGHEST) + psq_v


def _stage_a2qk_body(tb, nbw, nblk, n_cand, max_k,
                     s_ref, d_ref, cd_ref, cs_ref, tauq_ref, tauk_ref,
                     qkn_ref, q_ref, k_ref, pos_ref):
    t64, t16 = _thresholds_compact(cd_ref, cs_ref, tb, n_cand, max_k)
    tau_q = tauq_ref[...]
    tau_k = tauk_ref[...]

    def p3a(j, carry):
        gs_q, gm_q, gs_k, gm_k, pos_q = carry
        sl = pl.ds(j * nbw, nbw)
        s = s_ref[:, sl]
        d = d_ref[:, sl]
        sel = _sel_of(d, s, t64, t16)
        eq = _eg(s, tau_q, sel)
        ek = _eg(s, tau_k, sel)
        gs_q += jnp.sum(eq, axis=1, keepdims=True)
        gm_q = jnp.maximum(gm_q, jnp.max(eq, axis=1, keepdims=True))
        gs_k += jnp.sum(ek, axis=1, keepdims=True)
        gm_k = jnp.maximum(gm_k, jnp.max(ek, axis=1, keepdims=True))
        pos_q += jnp.sum(eq * d, axis=1, keepdims=True)
        return (gs_q, gm_q, gs_k, gm_k, pos_q)

    z = jnp.zeros((tb, 1), jnp.float32)
    gs_q, gm_q, gs_k, gm_k, pos_q = jax.lax.fori_loop(
        0, nblk, p3a, (z, z, z, z, z))
    sc_q = jnp.tanh(gm_q) / (gs_q + 1e-8)
    sc_k = jnp.tanh(gm_k) / (gs_k + 1e-8)

    q_ref[...] = jnp.zeros_like(q_ref)
    k_ref[...] = jnp.zeros_like(k_ref)

    def p3b(j, _):
        sl = pl.ds(j * nbw, nbw)
        s = s_ref[:, sl]
        sel = _sel_of(d_ref[:, sl], s, t64, t16)
        qkn = qkn_ref[sl, :]
        q_ref[...] += _dot(s * _eg(s, tau_q, sel), qkn, _DEFAULT)
        k_ref[...] += _dot(s * _eg(s, tau_k, sel), qkn, _DEFAULT)
        return 0

    jax.lax.fori_loop(0, nblk, p3b, 0)
    q_ref[...] *= sc_q
    k_ref[...] *= sc_k
    pos_ref[...] = jnp.sum(sc_q * pos_q)[None, None, None]


def _stage_a2v_body(tb, nbw, nblk, n_cand, max_k,
                    s_ref, d_ref, cd_ref, cs_ref, tauv_ref, vn_ref,
                    v_ref, pos_ref):
    t64, t16 = _thresholds_compact(cd_ref, cs_ref, tb, n_cand, max_k)
    tau_v = tauv_ref[...]

    def p3a(j, carry):
        gs_v, gm_v, pos_v = carry
        sl = pl.ds(j * nbw, nbw)
        s = s_ref[:, sl]
        d = d_ref[:, sl]
        sel = _sel_of(d, s, t64, t16)
        ev = _eg(s, tau_v, sel)
        gs_v += jnp.sum(ev, axis=1, keepdims=True)
        gm_v = jnp.maximum(gm_v, jnp.max(ev, axis=1, keepdims=True))
        pos_v += jnp.sum(ev * d, axis=1, keepdims=True)
        return (gs_v, gm_v, pos_v)

    z = jnp.zeros((tb, 1), jnp.float32)
    gs_v, gm_v, pos_v = jax.lax.fori_loop(0, nblk, p3a, (z, z, z))
    sc_v = jnp.tanh(gm_v) / (gs_v + 1e-8)

    v_ref[...] = jnp.zeros_like(v_ref)

    def p3b(j, _):
        sl = pl.ds(j * nbw, nbw)
        s = s_ref[:, sl]
        sel = _sel_of(d_ref[:, sl], s, t64, t16)
        v_ref[...] += _dot(s * _eg(s, tau_v, sel), vn_ref[sl, :], _DEFAULT)
        return 0

    jax.lax.fori_loop(0, nblk, p3b, 0)
    v_ref[...] *= sc_v
    pos_ref[...] = jnp.sum(sc_v * pos_v)[None, None, None]


_SC_INF = 3.0e38
_SC_CAP = 256


def _make_sc_collect(S, N, cap):
    NW = 32
    t_per_w = S // NW
    mesh = plsc.VectorSubcoreMesh(core_axis_name="c", subcore_axis_name="s")

    @functools.partial(
        pl.kernel, mesh=mesh,
        compiler_params=pltpu.CompilerParams(needs_layout_passes=False),
        out_type=[
            jax.ShapeDtypeStruct((S, cap), jnp.float32),
            jax.ShapeDtypeStruct((S, cap), jnp.float32),
            jax.ShapeDtypeStruct((S, cap), jnp.float32),
            jax.ShapeDtypeStruct((S, cap), jnp.float32),
        ],
        scratch_types=[
            pltpu.VMEM((N,), jnp.float32),
            pltpu.VMEM((N,), jnp.float32),
            pltpu.VMEM((cap,), jnp.float32),
            pltpu.VMEM((cap,), jnp.float32),
            pltpu.SemaphoreType.DMA,
            pltpu.SemaphoreType.DMA,
        ],
    )
    def sc_collect(dq_hbm, sq_hbm, dv_hbm, sv_hbm,
                   cdq_hbm, csq_hbm, cdv_hbm, csv_hbm,
                   drow, srow, cdrow, csrow, sem1, sem2):
        wid = lax.axis_index("s") * 2 + lax.axis_index("c")

        def one_pool(tok, d_hbm, s_hbm, cd_hbm, cs_hbm):
            cp1 = pltpu.async_copy(d_hbm.at[tok], drow, sem1)
            cp2 = pltpu.async_copy(s_hbm.at[tok], srow, sem2)
            cp1.wait()
            cp2.wait()

            def fold_group(g, bnd):
                def fold(i, m):
                    return jnp.minimum(m, drow[pl.ds((g * 16 + i) * 16, 16)])

                m = lax.fori_loop(0, 16, fold,
                                  jnp.full((16,), _SC_INF, jnp.float32))
                mn1 = jnp.min(m)
                m = jnp.where(m == mn1, jnp.float32(_SC_INF), m)
                mn2 = jnp.min(m)
                m = jnp.where(m == mn2, jnp.float32(_SC_INF), m)
                mn3 = jnp.min(m)
                m = jnp.where(m == mn3, jnp.float32(_SC_INF), m)
                return jnp.maximum(bnd, jnp.min(m))

            b = lax.fori_loop(0, 16, fold_group, jnp.float32(-_SC_INF))

            def fill(i, _):
                cdrow[pl.ds(i * 16, 16)] = jnp.full((16,), _SC_INF,
                                                    jnp.float32)
                return 0

            lax.fori_loop(0, cap // 16, fill, 0)

            def collect(i, ptr):
                d = drow[pl.ds(i * 16, 16)]
                m = d <= b
                cum = plsc.cumsum(m.astype(jnp.int32))
                idx = ptr + cum - 1
                m = m & (idx < cap)
                s = srow[pl.ds(i * 16, 16)]
                plsc.store_scatter(cdrow, [idx], d, mask=m)
                plsc.store_scatter(csrow, [idx], s, mask=m)
                return ptr + plsc.all_reduce_population_count(m)

            lax.fori_loop(0, N // 16, collect, jnp.zeros((16,), jnp.int32))
            pltpu.sync_copy(cdrow, cd_hbm.at[tok])
            pltpu.sync_copy(csrow, cs_hbm.at[tok])

        def per_token(t, _):
            tok = wid * t_per_w + t
            one_pool(tok, dq_hbm, sq_hbm, cdq_hbm, csq_hbm)
            one_pool(tok, dv_hbm, sv_hbm, cdv_hbm, csv_hbm)
            return 0

        lax.fori_loop(0, t_per_w, per_token, 0)

    return sc_collect


def _stage_b_body(scale, tbq, x_ref, k_ref, v_ref, o_ref):
    i = pl.program_id(1)
    q = x_ref[0]
    k = k_ref[0]
    v = v_ref[0]
    s = _dotT(q, k, _DEFAULT) * (1.0 / scale)
    row = jax.lax.broadcasted_iota(jnp.int32, s.shape, 0) + i * tbq
    col = jax.lax.broadcasted_iota(jnp.int32, s.shape, 1)
    s = jnp.where(col <= row, s, jnp.finfo(jnp.float32).min)
    m = jnp.max(s, axis=1, keepdims=True)
    p = jnp.exp(s - m)
    l = jnp.sum(p, axis=1, keepdims=True)
    o_ref[...] = (_dot(p, v, _DEFAULT) / l)[None]


def _stage_c_body(a_ref, w_ref, o_ref):
    o_ref[...] = _dot(a_ref[...], w_ref[...], _DEFAULT)


def kernel(x, qk_pos, v_pos, tau_Q, tau_K, tau_V, qk_neurons, v_neurons,
           npos_qk, npos_v, expand_O_kernel):
    B, S, D = x.shape
    N, _ = qk_neurons.shape
    P = qk_pos.shape[-1]
    n_heads = 16
    dh = D // n_heads
    max_k = 16
    n_cand = max_k * 4

    x2 = x.reshape(S, D)
    qkp = qk_pos.reshape(S, P)
    vp = v_pos.reshape(S, P)
    tq = tau_Q.reshape(S, 1)
    tk = tau_K.reshape(S, 1)
    tv = tau_V.reshape(S, 1)

    tb = 128
    nb = S // tb
    nbw = 512
    nblk = N // nbw

    tok = lambda i: (i, 0)
    full = lambda i: (0, 0)

    sqk, sv, dqk, dv = pl.pallas_call(
        _stage_a1_body,
        grid=(nblk, nb),
        in_specs=[
            pl.BlockSpec((tb, D), lambda j, i: (i, 0)),
            pl.BlockSpec((tb, P), lambda j, i: (i, 0)),
            pl.BlockSpec((tb, P), lambda j, i: (i, 0)),
            pl.BlockSpec((nbw, D), lambda j, i: (j, 0)),
            pl.BlockSpec((nbw, D), lambda j, i: (j, 0)),
            pl.BlockSpec((nbw, P), lambda j, i: (j, 0)),
            pl.BlockSpec((nbw, P), lambda j, i: (j, 0)),
        ],
        out_specs=[
            pl.BlockSpec((tb, nbw), lambda j, i: (i, j)),
            pl.BlockSpec((tb, nbw), lambda j, i: (i, j)),
            pl.BlockSpec((tb, nbw), lambda j, i: (i, j)),
            pl.BlockSpec((tb, nbw), lambda j, i: (i, j)),
        ],
        out_shape=[
            jax.ShapeDtypeStruct((S, N), jnp.float32),
            jax.ShapeDtypeStruct((S, N), jnp.float32),
            jax.ShapeDtypeStruct((S, N), jnp.float32),
            jax.ShapeDtypeStruct((S, N), jnp.float32),
        ],
    )(x2, qkp, vp, qk_neurons, v_neurons, npos_qk, npos_v)

    cap = _SC_CAP
    cdq, csq, cdv_c, csv_c = _make_sc_collect(S, N, cap)(dqk, sqk, dv, sv)

    q_buf, k_buf, pos_qk = pl.pallas_call(
        functools.partial(_stage_a2qk_body, tb, nbw, nblk, n_cand, max_k),
        grid=(nb,),
        in_specs=[
            pl.BlockSpec((tb, N), tok),
            pl.BlockSpec((tb, N), tok),
            pl.BlockSpec((tb, cap), tok),
            pl.BlockSpec((tb, cap), tok),
            pl.BlockSpec((tb, 1), tok),
            pl.BlockSpec((tb, 1), tok),
            pl.BlockSpec((N, D), full),
        ],
        out_specs=[
            pl.BlockSpec((tb, D), tok),
            pl.BlockSpec((tb, D), tok),
            pl.BlockSpec((1, 1, 1), lambda i: (i, 0, 0)),
        ],
        out_shape=[
            jax.ShapeDtypeStruct((S, D), jnp.float32),
            jax.ShapeDtypeStruct((S, D), jnp.float32),
            jax.ShapeDtypeStruct((nb, 1, 1), jnp.float32),
        ],
    )(sqk, dqk, cdq, csq, tq, tk, qk_neurons)

    v_buf, pos_v = pl.pallas_call(
        functools.partial(_stage_a2v_body, tb, nbw, nblk, n_cand, max_k),
        grid=(nb,),
        in_specs=[
            pl.BlockSpec((tb, N), tok),
            pl.BlockSpec((tb, N), tok),
            pl.BlockSpec((tb, cap), tok),
            pl.BlockSpec((tb, cap), tok),
            pl.BlockSpec((tb, 1), tok),
            pl.BlockSpec((N, D), full),
        ],
        out_specs=[
            pl.BlockSpec((tb, D), tok),
            pl.BlockSpec((1, 1, 1), lambda i: (i, 0, 0)),
        ],
        out_shape=[
            jax.ShapeDtypeStruct((S, D), jnp.float32),
            jax.ShapeDtypeStruct((nb, 1, 1), jnp.float32),
        ],
    )(sv, dv, cdv_c, csv_c, tv, v_neurons)

    tbq = 128
    nbq = S // tbq
    scale = float(dh) ** 0.5
    q3 = q_buf.reshape(S, n_heads, dh).transpose(1, 0, 2)
    k3 = k_buf.reshape(S, n_heads, dh).transpose(1, 0, 2)
    v3 = v_buf.reshape(S, n_heads, dh).transpose(1, 0, 2)
    attn = pl.pallas_call(
        functools.partial(_stage_b_body, scale, tbq),
        grid=(n_heads, nbq),
        in_specs=[
            pl.BlockSpec((1, tbq, dh), lambda h, i: (h, i, 0)),
            pl.BlockSpec((1, S, dh), lambda h, i: (h, 0, 0)),
            pl.BlockSpec((1, S, dh), lambda h, i: (h, 0, 0)),
        ],
        out_specs=pl.BlockSpec((1, tbq, dh), lambda h, i: (h, i, 0)),
        out_shape=jax.ShapeDtypeStruct((n_heads, S, dh), jnp.float32),
    )(q3, k3, v3)
    attn = attn.transpose(1, 0, 2).reshape(S, D)

    out = pl.pallas_call(
        _stage_c_body,
        grid=(nb,),
        in_specs=[
            pl.BlockSpec((tb, D), tok),
            pl.BlockSpec((D, D), full),
        ],
        out_specs=pl.BlockSpec((tb, D), tok),
        out_shape=jax.ShapeDtypeStruct((S, D), jnp.float32),
    )(attn, expand_O_kernel)

    pos_loss = (jnp.sum(pos_qk) + jnp.sum(pos_v)) / jnp.float32(S * n_cand)
    return out.reshape(B, S, D), pos_loss


# speedup vs baseline: 3.8186x; 2.5590x over previous
import functools

import jax
import jax.numpy as jnp
from jax import lax
from jax.experimental import pallas as pl
from jax.experimental.pallas import tpu as pltpu
from jax.experimental.pallas import tpu_sc as plsc

_HIGHEST = jax.lax.Precision.HIGHEST
_HIGH = jax.lax.Precision.HIGH
_DEFAULT = jax.lax.Precision.DEFAULT
_INT_MIN = -2147483648
_INT_MAX = 2147483647
_MAG = 0x7FFFFFFF


def _dotT(a, b, precision=_HIGHEST):
    return jax.lax.dot_general(a, b, (((1,), (1,)), ((), ())),
                               preferred_element_type=jnp.float32,
                               precision=precision)


def _dot(a, b, precision=_HIGHEST):
    return jax.lax.dot_general(a, b, (((1,), (0,)), ((), ())),
                               preferred_element_type=jnp.float32,
                               precision=precision)


def _monotone_int(x):
    b = jax.lax.bitcast_convert_type(x, jnp.int32)
    return jnp.where(b < 0, -(b & _MAG), b)


def _kth_smallest_blocks(key_block_fn, nblk, tb, w, k, lo0, hi0):
    kf = jnp.float32(k)
    ones_w = jnp.ones((w, 1), jnp.float32)

    def cond(carry):
        i, lo, hi = carry
        return (i < 32) & jnp.any(lo < hi)

    def body(carry):
        i, lo, hi = carry
        mid = (lo >> 1) + (hi >> 1) + (lo & hi & 1)

        def cnt_body(j, cnt):
            blk = (key_block_fn(j) <= mid).astype(jnp.float32)
            return cnt + _dot(blk, ones_w, _DEFAULT)

        cnt = jax.lax.fori_loop(0, nblk, cnt_body,
                                jnp.zeros((tb, 1), jnp.float32))
        ge = cnt >= kf
        return i + 1, jnp.where(ge, lo, mid + 1), jnp.where(ge, mid, hi)

    _, lo, hi = jax.lax.while_loop(cond, body, (jnp.int32(0), lo0, hi0))
    return hi


def _eg(s, tau, sel):
    raw = s - tau
    g = jnp.where(raw > 0, raw, 1e-8 * jnp.exp(jnp.minimum(raw, 0.0)))
    eg = jnp.exp(g) - 1.0
    return jnp.where(sel, eg, 0.0)


def _thresholds_compact(cd_ref, cs_ref, tb, n_cand, max_k):
    dkey = _monotone_int(cd_ref[...])
    cap = dkey.shape[1]
    dlo = jnp.min(dkey, axis=1, keepdims=True)
    dhi = jnp.max(dkey, axis=1, keepdims=True)
    t64 = _kth_smallest_blocks(lambda j: dkey, 1, tb, cap, n_cand, dlo, dhi)

    cand = dkey <= t64
    skey = jnp.where(cand, -_monotone_int(cs_ref[...]), jnp.int32(_INT_MAX))
    slo = jnp.min(skey, axis=1, keepdims=True)
    shi = jnp.max(jnp.where(cand, skey, _INT_MIN), axis=1, keepdims=True)
    t16 = _kth_smallest_blocks(lambda j: skey, 1, tb, cap, max_k, slo, shi)
    return t64, t16


def _sel_of(d_blk, s_blk, t64, t16):
    return (_monotone_int(d_blk) <= t64) & (-_monotone_int(s_blk) <= t16)


def _stage_a1_body(x_ref, qkpos_ref, vpos_ref, qkn_ref, vn_ref,
                   nposqk_ref, nposv_ref,
                   sqk_ref, sv_ref, dqk_ref, dv_ref):
    x = x_ref[...]
    qkpos = qkpos_ref[...]
    vpos = vpos_ref[...]
    qkn = qkn_ref[...]
    vn = vn_ref[...]
    npq = nposqk_ref[...]
    npv = nposv_ref[...]
    sqk_ref[...] = _dotT(x, qkn, _DEFAULT)
    sv_ref[...] = _dotT(x, vn, _DEFAULT)
    psq_qk = jnp.sum(qkpos * qkpos, axis=1, keepdims=True)
    psq_v = jnp.sum(vpos * vpos, axis=1, keepdims=True)
    nsq_q = jnp.sum(npq * npq, axis=1)[None, :]
    nsq_v = jnp.sum(npv * npv, axis=1)[None, :]
    dqk_ref[...] = nsq_q - 2.0 * _dotT(qkpos, npq, _HIGHEST) + psq_qk
    dv_ref[...] = nsq_v - 2.0 * _dotT(vpos, npv, _HIGHEST) + psq_v


def _stage_a2qk_body(tb, nbw, nblk, n_cand, max_k,
                     s_ref, d_ref, cd_ref, cs_ref, tauq_ref, tauk_ref,
                     qkn_ref, q_ref, k_ref, pos_ref):
    t64, t16 = _thresholds_compact(cd_ref, cs_ref, tb, n_cand, max_k)
    tau_q = tauq_ref[...]
    tau_k = tauk_ref[...]
    ones_w = jnp.ones((nbw, 1), jnp.float32)

    def p3a(j, carry):
        gs_q, gm_q, gs_k, gm_k, pos_q = carry
        sl = pl.ds(j * nbw, nbw)
        s = s_ref[:, sl]
        d = d_ref[:, sl]
        sel = _sel_of(d, s, t64, t16)
        eq = _eg(s, tau_q, sel)
        ek = _eg(s, tau_k, sel)
        gs_q += _dot(eq, ones_w, _HIGHEST)
        gm_q = jnp.maximum(gm_q, jnp.max(eq, axis=1, keepdims=True))
        gs_k += _dot(ek, ones_w, _HIGHEST)
        gm_k = jnp.maximum(gm_k, jnp.max(ek, axis=1, keepdims=True))
        pos_q += _dot(eq * d, ones_w, _HIGHEST)
        return (gs_q, gm_q, gs_k, gm_k, pos_q)

    z = jnp.zeros((tb, 1), jnp.float32)
    gs_q, gm_q, gs_k, gm_k, pos_q = jax.lax.fori_loop(
        0, nblk, p3a, (z, z, z, z, z))
    sc_q = jnp.tanh(gm_q) / (gs_q + 1e-8)
    sc_k = jnp.tanh(gm_k) / (gs_k + 1e-8)

    q_ref[...] = jnp.zeros_like(q_ref)
    k_ref[...] = jnp.zeros_like(k_ref)

    def p3b(j, _):
        sl = pl.ds(j * nbw, nbw)
        s = s_ref[:, sl]
        sel = _sel_of(d_ref[:, sl], s, t64, t16)
        qkn = qkn_ref[sl, :]
        q_ref[...] += _dot(s * _eg(s, tau_q, sel), qkn, _DEFAULT)
        k_ref[...] += _dot(s * _eg(s, tau_k, sel), qkn, _DEFAULT)
        return 0

    jax.lax.fori_loop(0, nblk, p3b, 0)
    q_ref[...] *= sc_q
    k_ref[...] *= sc_k
    pos_ref[...] = jnp.sum(sc_q * pos_q)[None, None, None]


def _stage_a2v_body(tb, nbw, nblk, n_cand, max_k,
                    s_ref, d_ref, cd_ref, cs_ref, tauv_ref, vn_ref,
                    v_ref, pos_ref):
    t64, t16 = _thresholds_compact(cd_ref, cs_ref, tb, n_cand, max_k)
    tau_v = tauv_ref[...]

    ones_w = jnp.ones((nbw, 1), jnp.float32)

    def p3a(j, carry):
        gs_v, gm_v, pos_v = carry
        sl = pl.ds(j * nbw, nbw)
        s = s_ref[:, sl]
        d = d_ref[:, sl]
        sel = _sel_of(d, s, t64, t16)
        ev = _eg(s, tau_v, sel)
        gs_v += _dot(ev, ones_w, _HIGHEST)
        gm_v = jnp.maximum(gm_v, jnp.max(ev, axis=1, keepdims=True))
        pos_v += _dot(ev * d, ones_w, _HIGHEST)
        return (gs_v, gm_v, pos_v)

    z = jnp.zeros((tb, 1), jnp.float32)
    gs_v, gm_v, pos_v = jax.lax.fori_loop(0, nblk, p3a, (z, z, z))
    sc_v = jnp.tanh(gm_v) / (gs_v + 1e-8)

    v_ref[...] = jnp.zeros_like(v_ref)

    def p3b(j, _):
        sl = pl.ds(j * nbw, nbw)
        s = s_ref[:, sl]
        sel = _sel_of(d_ref[:, sl], s, t64, t16)
        v_ref[...] += _dot(s * _eg(s, tau_v, sel), vn_ref[sl, :], _DEFAULT)
        return 0

    jax.lax.fori_loop(0, nblk, p3b, 0)
    v_ref[...] *= sc_v
    pos_ref[...] = jnp.sum(sc_v * pos_v)[None, None, None]


_SC_INF = 3.0e38
_SC_CAP = 256


def _make_sc_collect(S, N, cap):
    NW = 32
    t_per_w = S // NW
    mesh = plsc.VectorSubcoreMesh(core_axis_name="c", subcore_axis_name="s")

    @functools.partial(
        pl.kernel, mesh=mesh,
        compiler_params=pltpu.CompilerParams(needs_layout_passes=False),
        out_type=[
            jax.ShapeDtypeStruct((S, cap), jnp.float32),
            jax.ShapeDtypeStruct((S, cap), jnp.float32),
            jax.ShapeDtypeStruct((S, cap), jnp.float32),
            jax.ShapeDtypeStruct((S, cap), jnp.float32),
        ],
        scratch_types=[
            pltpu.VMEM((N,), jnp.float32),
            pltpu.VMEM((N,), jnp.float32),
            pltpu.VMEM((cap,), jnp.float32),
            pltpu.VMEM((cap,), jnp.float32),
            pltpu.SemaphoreType.DMA,
            pltpu.SemaphoreType.DMA,
        ],
    )
    def sc_collect(dq_hbm, sq_hbm, dv_hbm, sv_hbm,
                   cdq_hbm, csq_hbm, cdv_hbm, csv_hbm,
                   drow, srow, cdrow, csrow, sem1, sem2):
        wid = lax.axis_index("s") * 2 + lax.axis_index("c")

        def one_pool(tok, d_hbm, s_hbm, cd_hbm, cs_hbm):
            cp1 = pltpu.async_copy(d_hbm.at[tok], drow, sem1)
            cp2 = pltpu.async_copy(s_hbm.at[tok], srow, sem2)
            cp1.wait()
            cp2.wait()

            def fold_group(g, bnd):
                def fold(i, m):
                    return jnp.minimum(m, drow[pl.ds((g * 16 + i) * 16, 16)])

                m = lax.fori_loop(0, 16, fold,
                                  jnp.full((16,), _SC_INF, jnp.float32))
                mn1 = jnp.min(m)
                m = jnp.where(m == mn1, jnp.float32(_SC_INF), m)
                mn2 = jnp.min(m)
                m = jnp.where(m == mn2, jnp.float32(_SC_INF), m)
                mn3 = jnp.min(m)
                m = jnp.where(m == mn3, jnp.float32(_SC_INF), m)
                return jnp.maximum(bnd, jnp.min(m))

            b = lax.fori_loop(0, 16, fold_group, jnp.float32(-_SC_INF))

            def fill(i, _):
                cdrow[pl.ds(i * 16, 16)] = jnp.full((16,), _SC_INF,
                                                    jnp.float32)
                return 0

            lax.fori_loop(0, cap // 16, fill, 0)

            def collect(i, ptr):
                d = drow[pl.ds(i * 16, 16)]
                m = d <= b
                cum = plsc.cumsum(m.astype(jnp.int32))
                idx = ptr + cum - 1
                m = m & (idx < cap)
                s = srow[pl.ds(i * 16, 16)]
                plsc.store_scatter(cdrow, [idx], d, mask=m)
                plsc.store_scatter(csrow, [idx], s, mask=m)
                return ptr + plsc.all_reduce_population_count(m)

            lax.fori_loop(0, N // 16, collect, jnp.zeros((16,), jnp.int32))
            pltpu.sync_copy(cdrow, cd_hbm.at[tok])
            pltpu.sync_copy(csrow, cs_hbm.at[tok])

        def per_token(t, _):
            tok = wid * t_per_w + t
            one_pool(tok, dq_hbm, sq_hbm, cdq_hbm, csq_hbm)
            one_pool(tok, dv_hbm, sv_hbm, cdv_hbm, csv_hbm)
            return 0

        lax.fori_loop(0, t_per_w, per_token, 0)

    return sc_collect


def _stage_b_body(scale, tbq, x_ref, k_ref, v_ref, o_ref):
    i = pl.program_id(1)
    q = x_ref[0]
    k = k_ref[0]
    v = v_ref[0]
    s = _dotT(q, k, _DEFAULT) * (1.0 / scale)
    row = jax.lax.broadcasted_iota(jnp.int32, s.shape, 0) + i * tbq
    col = jax.lax.broadcasted_iota(jnp.int32, s.shape, 1)
    s = jnp.where(col <= row, s, jnp.finfo(jnp.float32).min)
    m = jnp.max(s, axis=1, keepdims=True)
    p = jnp.exp(s - m)
    l = _dot(p, jnp.ones((p.shape[1], 1), jnp.float32), _HIGHEST)
    o_ref[...] = (_dot(p, v, _DEFAULT) / l)[None]


def _stage_c_body(a_ref, w_ref, o_ref):
    o_ref[...] = _dot(a_ref[...], w_ref[...], _DEFAULT)


def kernel(x, qk_pos, v_pos, tau_Q, tau_K, tau_V, qk_neurons, v_neurons,
           npos_qk, npos_v, expand_O_kernel):
    B, S, D = x.shape
    N, _ = qk_neurons.shape
    P = qk_pos.shape[-1]
    n_heads = 16
    dh = D // n_heads
    max_k = 16
    n_cand = max_k * 4

    x2 = x.reshape(S, D)
    qkp = qk_pos.reshape(S, P)
    vp = v_pos.reshape(S, P)
    tq = tau_Q.reshape(S, 1)
    tk = tau_K.reshape(S, 1)
    tv = tau_V.reshape(S, 1)

    tb = 128
    nb = S // tb
    nbw = 512
    nblk = N // nbw

    tok = lambda i: (i, 0)
    full = lambda i: (0, 0)

    sqk, sv, dqk, dv = pl.pallas_call(
        _stage_a1_body,
        grid=(nblk, nb),
        in_specs=[
            pl.BlockSpec((tb, D), lambda j, i: (i, 0)),
            pl.BlockSpec((tb, P), lambda j, i: (i, 0)),
            pl.BlockSpec((tb, P), lambda j, i: (i, 0)),
            pl.BlockSpec((nbw, D), lambda j, i: (j, 0)),
            pl.BlockSpec((nbw, D), lambda j, i: (j, 0)),
            pl.BlockSpec((nbw, P), lambda j, i: (j, 0)),
            pl.BlockSpec((nbw, P), lambda j, i: (j, 0)),
        ],
        out_specs=[
            pl.BlockSpec((tb, nbw), lambda j, i: (i, j)),
            pl.BlockSpec((tb, nbw), lambda j, i: (i, j)),
            pl.BlockSpec((tb, nbw), lambda j, i: (i, j)),
            pl.BlockSpec((tb, nbw), lambda j, i: (i, j)),
        ],
        out_shape=[
            jax.ShapeDtypeStruct((S, N), jnp.float32),
            jax.ShapeDtypeStruct((S, N), jnp.float32),
            jax.ShapeDtypeStruct((S, N), jnp.float32),
            jax.ShapeDtypeStruct((S, N), jnp.float32),
        ],
    )(x2, qkp, vp, qk_neurons, v_neurons, npos_qk, npos_v)

    cap = _SC_CAP
    cdq, csq, cdv_c, csv_c = _make_sc_collect(S, N, cap)(dqk, sqk, dv, sv)

    q_buf, k_buf, pos_qk = pl.pallas_call(
        functools.partial(_stage_a2qk_body, tb, nbw, nblk, n_cand, max_k),
        grid=(nb,),
        in_specs=[
            pl.BlockSpec((tb, N), tok),
            pl.BlockSpec((tb, N), tok),
            pl.BlockSpec((tb, cap), tok),
            pl.BlockSpec((tb, cap), tok),
            pl.BlockSpec((tb, 1), tok),
            pl.BlockSpec((tb, 1), tok),
            pl.BlockSpec((N, D), full),
        ],
        out_specs=[
            pl.BlockSpec((tb, D), tok),
            pl.BlockSpec((tb, D), tok),
            pl.BlockSpec((1, 1, 1), lambda i: (i, 0, 0)),
        ],
        out_shape=[
            jax.ShapeDtypeStruct((S, D), jnp.float32),
            jax.ShapeDtypeStruct((S, D), jnp.float32),
            jax.ShapeDtypeStruct((nb, 1, 1), jnp.float32),
        ],
    )(sqk, dqk, cdq, csq, tq, tk, qk_neurons)

    v_buf, pos_v = pl.pallas_call(
        functools.partial(_stage_a2v_body, tb, nbw, nblk, n_cand, max_k),
        grid=(nb,),
        in_specs=[
            pl.BlockSpec((tb, N), tok),
            pl.BlockSpec((tb, N), tok),
            pl.BlockSpec((tb, cap), tok),
            pl.BlockSpec((tb, cap), tok),
            pl.BlockSpec((tb, 1), tok),
            pl.BlockSpec((N, D), full),
        ],
        out_specs=[
            pl.BlockSpec((tb, D), tok),
            pl.BlockSpec((1, 1, 1), lambda i: (i, 0, 0)),
        ],
        out_shape=[
            jax.ShapeDtypeStruct((S, D), jnp.float32),
            jax.ShapeDtypeStruct((nb, 1, 1), jnp.float32),
        ],
    )(sv, dv, cdv_c, csv_c, tv, v_neurons)

    tbq = 128
    nbq = S // tbq
    scale = float(dh) ** 0.5
    q3 = q_buf.reshape(S, n_heads, dh).transpose(1, 0, 2)
    k3 = k_buf.reshape(S, n_heads, dh).transpose(1, 0, 2)
    v3 = v_buf.reshape(S, n_heads, dh).transpose(1, 0, 2)
    attn = pl.pallas_call(
        functools.partial(_stage_b_body, scale, tbq),
        grid=(n_heads, nbq),
        in_specs=[
            pl.BlockSpec((1, tbq, dh), lambda h, i: (h, i, 0)),
            pl.BlockSpec((1, S, dh), lambda h, i: (h, 0, 0)),
            pl.BlockSpec((1, S, dh), lambda h, i: (h, 0, 0)),
        ],
        out_specs=pl.BlockSpec((1, tbq, dh), lambda h, i: (h, i, 0)),
        out_shape=jax.ShapeDtypeStruct((n_heads, S, dh), jnp.float32),
    )(q3, k3, v3)
    attn = attn.transpose(1, 0, 2).reshape(S, D)

    out = pl.pallas_call(
        _stage_c_body,
        grid=(nb,),
        in_specs=[
            pl.BlockSpec((tb, D), tok),
            pl.BlockSpec((D, D), full),
        ],
        out_specs=pl.BlockSpec((tb, D), tok),
        out_shape=jax.ShapeDtypeStruct((S, D), jnp.float32),
    )(attn, expand_O_kernel)

    pos_loss = (jnp.sum(pos_qk) + jnp.sum(pos_v)) / jnp.float32(S * n_cand)
    return out.reshape(B, S, D), pos_loss
